# baseline (speedup 1.0000x reference)
import functools

import jax
import jax.numpy as jnp
from jax import lax
from jax.experimental import pallas as pl
from jax.experimental.pallas import tpu as pltpu
from jax.experimental.pallas import tpu_sc as plsc

NUM_FEATURES = 26
VOCAB = 100000
EMB = 32
B = 16384
H1 = 128
H2 = 64
EPS = 1e-5

NC = 2
NS = 16
NW = NC * NS
BPW = B // NW
SUB = 128
NSUB = BPW // SUB


GRAN = 16
GPR = EMB // GRAN
GB = B * GPR
GGW = GB // NW
NSUBG = GGW // SUB


def _sc_gather(feats, tables):
    tabs16 = [t.reshape(VOCAB * GPR, GRAN) for t in tables]
    feats_all = jnp.stack(feats)
    idx2 = jnp.stack([feats_all * GPR, feats_all * GPR + 1], axis=-1)
    idx2 = idx2.reshape(NUM_FEATURES, GB // SUB, SUB)

    mesh = plsc.VectorSubcoreMesh(
        core_axis_name="c", subcore_axis_name="s", num_cores=NC, num_subcores=NS
    )

    @functools.partial(
        pl.kernel,
        out_type=jax.ShapeDtypeStruct((NUM_FEATURES, GB, GRAN), jnp.float32),
        mesh=mesh,
        scratch_types=[
            pltpu.VMEM((NSUBG, SUB), jnp.int32),
            pltpu.VMEM((GGW, GRAN), jnp.float32),
            pltpu.SemaphoreType.DMA,
        ],
        compiler_params=pltpu.CompilerParams(use_tc_tiling_on_sc=False),
    )
    def gather_kernel(idx_hbm, *rest):
        tabs = rest[:NUM_FEATURES]
        out_hbm, idx_v, rows_v, sem = rest[NUM_FEATURES:]
        wid = lax.axis_index("s") * NC + lax.axis_index("c")
        base = wid * GGW
        row0 = wid * NSUBG
        for i in range(NUM_FEATURES):
            pltpu.sync_copy(idx_hbm.at[i, pl.ds(row0, NSUBG)], idx_v)

            def sub(j, carry, tab=tabs[i]):
                pltpu.async_copy(
                    tab.at[idx_v.at[j]], rows_v.at[pl.ds(j * SUB, SUB)], sem
                ).wait()
                return carry

            lax.fori_loop(0, NSUBG, sub, 0)
            pltpu.sync_copy(rows_v, out_hbm.at[i, pl.ds(base, GGW)])

    out = gather_kernel(idx2, *tabs16)
    return out.reshape(NUM_FEATURES, B, EMB)


BLK = 256


def _mlp_body(x_ref, w1_ref, p1_ref, w2_ref, p2_ref, o_ref):
    acc = jnp.zeros((BLK, H1), dtype=jnp.float32)
    for i in range(NUM_FEATURES):
        acc += jnp.dot(
            x_ref[i],
            w1_ref[i],
            preferred_element_type=jnp.float32,
            precision=lax.Precision.HIGHEST,
        )
    scale1 = p1_ref[1] * lax.rsqrt(p1_ref[4] + EPS)
    shift1 = p1_ref[2] + (p1_ref[0] - p1_ref[3]) * scale1
    h = jnp.maximum(acc * scale1 + shift1, 0.0)
    acc2 = jnp.dot(
        h,
        w2_ref[...],
        preferred_element_type=jnp.float32,
        precision=lax.Precision.HIGHEST,
    )
    scale2 = p2_ref[1] * lax.rsqrt(p2_ref[4] + EPS)
    shift2 = p2_ref[2] + (p2_ref[0] - p2_ref[3]) * scale2
    o_ref[...] = jnp.maximum(acc2 * scale2 + shift2, 0.0)


def _tc_mlp(x26, w1r, p1, w2, p2):
    grid = (B // BLK,)
    return pl.pallas_call(
        _mlp_body,
        grid=grid,
        in_specs=[
            pl.BlockSpec((NUM_FEATURES, BLK, EMB), lambda j: (0, j, 0)),
            pl.BlockSpec((NUM_FEATURES, EMB, H1), lambda j: (0, 0, 0)),
            pl.BlockSpec((5, H1), lambda j: (0, 0)),
            pl.BlockSpec((H1, H2), lambda j: (0, 0)),
            pl.BlockSpec((5, H2), lambda j: (0, 0)),
        ],
        out_specs=pl.BlockSpec((BLK, H2), lambda j: (j, 0)),
        out_shape=jax.ShapeDtypeStruct((B, H2), jnp.float32),
    )(x26, w1r, p1, w2, p2)


def kernel(feat_0, feat_1, feat_2, feat_3, feat_4, feat_5, feat_6, feat_7, feat_8, feat_9, feat_10, feat_11, feat_12, feat_13, feat_14, feat_15, feat_16, feat_17, feat_18, feat_19, feat_20, feat_21, feat_22, feat_23, feat_24, feat_25, table_0, table_1, table_2, table_3, table_4, table_5, table_6, table_7, table_8, table_9, table_10, table_11, table_12, table_13, table_14, table_15, table_16, table_17, table_18, table_19, table_20, table_21, table_22, table_23, table_24, table_25, W1, b1, gamma1, beta1, rm1, rv1, W2, b2, gamma2, beta2, rm2, rv2):
    feats = [feat_0, feat_1, feat_2, feat_3, feat_4, feat_5, feat_6, feat_7,
             feat_8, feat_9, feat_10, feat_11, feat_12, feat_13, feat_14,
             feat_15, feat_16, feat_17, feat_18, feat_19, feat_20, feat_21,
             feat_22, feat_23, feat_24, feat_25]
    tables = [table_0, table_1, table_2, table_3, table_4, table_5, table_6,
              table_7, table_8, table_9, table_10, table_11, table_12,
              table_13, table_14, table_15, table_16, table_17, table_18,
              table_19, table_20, table_21, table_22, table_23, table_24,
              table_25]
    x26 = _sc_gather(feats, tables)
    w1r = W1.reshape(NUM_FEATURES, EMB, H1)
    p1 = jnp.stack([b1, gamma1, beta1, rm1, rv1])
    p2 = jnp.stack([b2, gamma2, beta2, rm2, rv2])
    return _tc_mlp(x26, w1r, p1, w2=W2, p2=p2)


# speedup vs baseline: 1.4926x; 1.4926x over previous
import functools

import jax
import jax.numpy as jnp
from jax import lax
from jax.experimental import pallas as pl
from jax.experimental.pallas import tpu as pltpu
from jax.experimental.pallas import tpu_sc as plsc

NUM_FEATURES = 26
VOCAB = 100000
EMB = 32
B = 16384
H1 = 128
H2 = 64
EPS = 1e-5

NC = 2
NS = 16
NW = NC * NS
SUB = 128
GRAN = 16
GPR = EMB // GRAN
GB = B * GPR
GGW = GB // NW
NSUBG = GGW // SUB


def _sc_gather(feats, tables):
    tabs16 = [t.reshape(VOCAB * GPR, GRAN) for t in tables]
    feats_all = jnp.stack(feats)
    idx2 = jnp.stack([feats_all * GPR, feats_all * GPR + 1], axis=-1)
    idx2 = idx2.reshape(NUM_FEATURES, GB // SUB, SUB)

    mesh = plsc.VectorSubcoreMesh(
        core_axis_name="c", subcore_axis_name="s", num_cores=NC, num_subcores=NS
    )

    @functools.partial(
        pl.kernel,
        out_type=jax.ShapeDtypeStruct(
            (NUM_FEATURES, NW, NSUBG, SUB, GRAN), jnp.float32
        ),
        mesh=mesh,
        scratch_types=[
            pltpu.VMEM((NSUBG, SUB), jnp.int32),
            pltpu.VMEM((NSUBG, SUB, GRAN), jnp.float32),
            pltpu.SemaphoreType.DMA,
        ],
        compiler_params=pltpu.CompilerParams(use_tc_tiling_on_sc=False),
    )
    def gather_kernel(idx_hbm, *rest):
        tabs = rest[:NUM_FEATURES]
        out_hbm, idx_v, rows_v, sem = rest[NUM_FEATURES:]
        wid = lax.axis_index("s") * NC + lax.axis_index("c")
        row0 = wid * NSUBG
        for i in range(NUM_FEATURES):
            pltpu.sync_copy(idx_hbm.at[i, pl.ds(row0, NSUBG)], idx_v)

            def sub(j, carry, tab=tabs[i]):
                pltpu.async_copy(tab.at[idx_v.at[j]], rows_v.at[j], sem).wait()
                return carry

            lax.fori_loop(0, NSUBG, sub, 0)
            pltpu.sync_copy(rows_v, out_hbm.at[i, wid])

    return gather_kernel(idx2, *tabs16)


PACK = 4
PB = B // PACK
BLKP = 128


def _mlp_body(x_ref, w1_ref, p1_ref, w2_ref, p2_ref, o_ref):
    scale1 = p1_ref[1] * lax.rsqrt(p1_ref[4] + EPS)
    shift1 = p1_ref[2] + (p1_ref[0] - p1_ref[3]) * scale1
    scale2 = p2_ref[1] * lax.rsqrt(p2_ref[4] + EPS)
    shift2 = p2_ref[2] + (p2_ref[0] - p2_ref[3]) * scale2
    for q in range(PACK):
        acc = jnp.zeros((BLKP, H1), dtype=jnp.float32)
        for i in range(NUM_FEATURES):
            acc += jnp.dot(
                x_ref[i][:, EMB * q:EMB * (q + 1)],
                w1_ref[i],
                preferred_element_type=jnp.float32,
                precision=lax.Precision.HIGHEST,
            )
        h = jnp.maximum(acc * scale1 + shift1, 0.0)
        h2 = jnp.dot(
            h,
            w2_ref[...],
            preferred_element_type=jnp.float32,
            precision=lax.Precision.HIGHEST,
        )
        o_ref[:, H2 * q:H2 * (q + 1)] = jnp.maximum(h2 * scale2 + shift2, 0.0)


def _tc_mlp(xp, w1r, p1, w2, p2):
    return pl.pallas_call(
        _mlp_body,
        grid=(PB // BLKP,),
        in_specs=[
            pl.BlockSpec((NUM_FEATURES, BLKP, PACK * EMB), lambda j: (0, j, 0)),
            pl.BlockSpec((NUM_FEATURES, EMB, H1), lambda j: (0, 0, 0)),
            pl.BlockSpec((5, H1), lambda j: (0, 0)),
            pl.BlockSpec((H1, H2), lambda j: (0, 0)),
            pl.BlockSpec((5, H2), lambda j: (0, 0)),
        ],
        out_specs=pl.BlockSpec((BLKP, PACK * H2), lambda j: (j, 0)),
        out_shape=jax.ShapeDtypeStruct((PB, PACK * H2), jnp.float32),
    )(xp, w1r, p1, w2, p2)


def kernel(feat_0, feat_1, feat_2, feat_3, feat_4, feat_5, feat_6, feat_7, feat_8, feat_9, feat_10, feat_11, feat_12, feat_13, feat_14, feat_15, feat_16, feat_17, feat_18, feat_19, feat_20, feat_21, feat_22, feat_23, feat_24, feat_25, table_0, table_1, table_2, table_3, table_4, table_5, table_6, table_7, table_8, table_9, table_10, table_11, table_12, table_13, table_14, table_15, table_16, table_17, table_18, table_19, table_20, table_21, table_22, table_23, table_24, table_25, W1, b1, gamma1, beta1, rm1, rv1, W2, b2, gamma2, beta2, rm2, rv2):
    feats = [feat_0, feat_1, feat_2, feat_3, feat_4, feat_5, feat_6, feat_7,
             feat_8, feat_9, feat_10, feat_11, feat_12, feat_13, feat_14,
             feat_15, feat_16, feat_17, feat_18, feat_19, feat_20, feat_21,
             feat_22, feat_23, feat_24, feat_25]
    tables = [table_0, table_1, table_2, table_3, table_4, table_5, table_6,
              table_7, table_8, table_9, table_10, table_11, table_12,
              table_13, table_14, table_15, table_16, table_17, table_18,
              table_19, table_20, table_21, table_22, table_23, table_24,
              table_25]
    xp = _sc_gather(feats, tables).reshape(NUM_FEATURES, PB, PACK * EMB)
    w1r = W1.reshape(NUM_FEATURES, EMB, H1)
    p1 = jnp.stack([b1, gamma1, beta1, rm1, rv1])
    p2 = jnp.stack([b2, gamma2, beta2, rm2, rv2])
    out = _tc_mlp(xp, w1r, p1, W2, p2)
    return out.reshape(B, H2)


# speedup vs baseline: 1.9830x; 1.3286x over previous
import functools

import jax
import jax.numpy as jnp
from jax import lax
from jax.experimental import pallas as pl
from jax.experimental.pallas import tpu as pltpu
from jax.experimental.pallas import tpu_sc as plsc

NUM_FEATURES = 26
VOCAB = 100000
EMB = 32
B = 16384
H1 = 128
H2 = 64
EPS = 1e-5

NC = 2
NS = 16
NW = NC * NS
BPW = B // NW
SUB = 128
NSUB = BPW // SUB


def _sc_gather(feats, tables):
    idx_all = jnp.stack(feats).reshape(NUM_FEATURES, B // SUB, SUB)

    mesh = plsc.VectorSubcoreMesh(
        core_axis_name="c", subcore_axis_name="s", num_cores=NC, num_subcores=NS
    )

    @functools.partial(
        pl.kernel,
        out_type=jax.ShapeDtypeStruct(
            (NUM_FEATURES, NW, NSUB, SUB, EMB), jnp.float32
        ),
        mesh=mesh,
        scratch_types=[
            pltpu.VMEM((NUM_FEATURES, NSUB, SUB), jnp.int32),
            pltpu.VMEM((2, NSUB, SUB, EMB), jnp.float32),
            pltpu.SemaphoreType.DMA,
            pltpu.SemaphoreType.DMA,
            pltpu.SemaphoreType.DMA,
        ],
        compiler_params=pltpu.CompilerParams(use_tc_tiling_on_sc=False),
    )
    def gather_kernel(idx_hbm, *rest):
        tabs = rest[:NUM_FEATURES]
        out_hbm, idx_v, rows_v, gsem, wsem0, wsem1 = rest[NUM_FEATURES:]
        wsems = (wsem0, wsem1)
        wid = lax.axis_index("s") * NC + lax.axis_index("c")
        row0 = wid * NSUB
        pltpu.sync_copy(idx_hbm.at[:, pl.ds(row0, NSUB)], idx_v)
        for i in range(NUM_FEATURES):
            buf = i % 2
            if i >= 2:
                pltpu.make_async_copy(
                    rows_v.at[buf], out_hbm.at[i - 2, wid], wsems[buf]
                ).wait()

            def sub(j, carry, tab=tabs[i], buf=buf):
                pltpu.async_copy(
                    tab.at[idx_v.at[i, j]], rows_v.at[buf, j], gsem
                ).wait()
                return carry

            lax.fori_loop(0, NSUB, sub, 0)
            pltpu.async_copy(rows_v.at[buf], out_hbm.at[i, wid], wsems[buf]).start()
        for i in (NUM_FEATURES - 2, NUM_FEATURES - 1):
            buf = i % 2
            pltpu.make_async_copy(
                rows_v.at[buf], out_hbm.at[i, wid], wsems[buf]
            ).wait()

    return gather_kernel(idx_all, *tables)


PACK = 4
PB = B // PACK
BLKP = 128


def _mlp_body(x_ref, w1_ref, p1_ref, w2_ref, p2_ref, o_ref):
    scale1 = p1_ref[1] * lax.rsqrt(p1_ref[4] + EPS)
    shift1 = p1_ref[2] + (p1_ref[0] - p1_ref[3]) * scale1
    scale2 = p2_ref[1] * lax.rsqrt(p2_ref[4] + EPS)
    shift2 = p2_ref[2] + (p2_ref[0] - p2_ref[3]) * scale2
    acc = jnp.zeros((BLKP, PACK * H1), dtype=jnp.float32)
    for i in range(NUM_FEATURES):
        acc += jnp.dot(
            x_ref[i], w1_ref[i], preferred_element_type=jnp.float32
        )
    h = jnp.maximum(acc * scale1 + shift1, 0.0)
    h2 = jnp.dot(h, w2_ref[...], preferred_element_type=jnp.float32)
    o_ref[...] = jnp.maximum(h2 * scale2 + shift2, 0.0)


def _tc_mlp(xp, w1bd, p1t, w2bd, p2t):
    return pl.pallas_call(
        _mlp_body,
        grid=(PB // BLKP,),
        in_specs=[
            pl.BlockSpec((NUM_FEATURES, BLKP, PACK * EMB), lambda j: (0, j, 0)),
            pl.BlockSpec((NUM_FEATURES, PACK * EMB, PACK * H1),
                         lambda j: (0, 0, 0)),
            pl.BlockSpec((5, PACK * H1), lambda j: (0, 0)),
            pl.BlockSpec((PACK * H1, PACK * H2), lambda j: (0, 0)),
            pl.BlockSpec((5, PACK * H2), lambda j: (0, 0)),
        ],
        out_specs=pl.BlockSpec((BLKP, PACK * H2), lambda j: (j, 0)),
        out_shape=jax.ShapeDtypeStruct((PB, PACK * H2), jnp.float32),
    )(xp, w1bd, p1t, w2bd, p2t)


def _block_diag4(w):
    k, n = w.shape
    out = jnp.zeros((PACK, k, PACK, n), dtype=w.dtype)
    idx = jnp.arange(PACK)
    out = out.at[idx, :, idx, :].set(jnp.broadcast_to(w, (PACK, k, n)))
    return out.reshape(PACK * k, PACK * n)


def kernel(feat_0, feat_1, feat_2, feat_3, feat_4, feat_5, feat_6, feat_7, feat_8, feat_9, feat_10, feat_11, feat_12, feat_13, feat_14, feat_15, feat_16, feat_17, feat_18, feat_19, feat_20, feat_21, feat_22, feat_23, feat_24, feat_25, table_0, table_1, table_2, table_3, table_4, table_5, table_6, table_7, table_8, table_9, table_10, table_11, table_12, table_13, table_14, table_15, table_16, table_17, table_18, table_19, table_20, table_21, table_22, table_23, table_24, table_25, W1, b1, gamma1, beta1, rm1, rv1, W2, b2, gamma2, beta2, rm2, rv2):
    feats = [feat_0, feat_1, feat_2, feat_3, feat_4, feat_5, feat_6, feat_7,
             feat_8, feat_9, feat_10, feat_11, feat_12, feat_13, feat_14,
             feat_15, feat_16, feat_17, feat_18, feat_19, feat_20, feat_21,
             feat_22, feat_23, feat_24, feat_25]
    tables = [table_0, table_1, table_2, table_3, table_4, table_5, table_6,
              table_7, table_8, table_9, table_10, table_11, table_12,
              table_13, table_14, table_15, table_16, table_17, table_18,
              table_19, table_20, table_21, table_22, table_23, table_24,
              table_25]
    xp = _sc_gather(feats, tables).reshape(NUM_FEATURES, PB, PACK * EMB)
    w1r = W1.reshape(NUM_FEATURES, EMB, H1)
    w1bd = jax.vmap(_block_diag4)(w1r)
    w2bd = _block_diag4(W2)
    p1t = jnp.tile(jnp.stack([b1, gamma1, beta1, rm1, rv1]), (1, PACK))
    p2t = jnp.tile(jnp.stack([b2, gamma2, beta2, rm2, rv2]), (1, PACK))
    out = _tc_mlp(xp, w1bd, p1t, w2bd, p2t)
    return out.reshape(B, H2)
